# baseline (speedup 1.0000x reference)
import dataclasses
import functools

import jax
import jax.numpy as jnp
from jax import lax
from jax.experimental import pallas as pl
from jax.experimental.pallas import tpu as pltpu
from jax.experimental.pallas import tpu_sc as plsc

N_NODES = 10000
N_EDGES = 320000
D = 128
N_GRAPHS = 200
K = 30
KS = 5

NC = 2
NS = 16
NPT = N_NODES // (NC * NS) + (16 - (N_NODES // (NC * NS)) % 16) % 16
PP = 6144
ZB = 96

_mesh = plsc.VectorSubcoreMesh(core_axis_name="c", subcore_axis_name="s")

_sc_params = pltpu.CompilerParams()
for _f, _v in (("needs_layout_passes", False), ("use_tc_tiling_on_sc", False)):
    if _f in pltpu.CompilerParams.__dataclass_fields__:
        _sc_params = dataclasses.replace(_sc_params, **{_f: _v})


def _sel_body(score_hbm, ngi_hbm, st_hbm, h_hbm, out_hbm,
              score_v, ngi_v, st_v, hbuf, zbuf, didx_v, sem):
    c = lax.axis_index("c")
    s = lax.axis_index("s")
    wid = c * NS + s
    pltpu.sync_copy(score_hbm, score_v)
    pltpu.sync_copy(ngi_hbm, ngi_v)
    pltpu.sync_copy(st_hbm, st_v)

    @pl.loop(0, ZB)
    def _(i):
        for cc in range(D // 16):
            zbuf[i, pl.ds(cc * 16, 16)] = jnp.zeros((16,), jnp.float32)

    half = c * PP
    for k in range(PP // NS // ZB):
        pltpu.sync_copy(zbuf, out_hbm.at[pl.ds(half + (s * (PP // NS)) + k * ZB, ZB)])
    plsc.subcore_barrier()

    nbase = wid * NPT
    for k in range(NPT // 80):
        @pl.when(nbase + k * 80 < N_NODES)
        def _():
            pltpu.sync_copy(h_hbm.at[pl.ds(nbase + k * 80, 80)],
                            hbuf.at[pl.ds(k * 80, 80)])

    iota = lax.iota(jnp.int32, 16)
    trash = half + PP - 1

    @pl.loop(0, NPT // 16)
    def _(l):
        gbase = nbase + l * 16

        @pl.when(gbase < N_NODES)
        def _():
            @pl.loop(0, 16)
            def _(ll):
                ni = gbase + ll
                nsplat = jnp.full((16,), ni, jnp.int32)
                s_i = plsc.load_gather(score_v, [nsplat])
                g_i = plsc.load_gather(ngi_v, [nsplat])
                st_i = plsc.load_gather(st_v, [g_i])
                en_i = plsc.load_gather(st_v, [g_i + 1])
                st_s = jnp.max(st_i)
                q0 = st_s // 16
                nch = (jnp.max(en_i) + 15) // 16 - q0

                def body(q, racc):
                    off = (q0 + q) * 16
                    ch = score_v[pl.ds(off, 16)]
                    vidx = iota + off
                    valid = (vidx >= st_i) & (vidx < en_i)
                    win = (ch > s_i) | ((ch == s_i) & (vidx < nsplat))
                    cnt = plsc.all_reduce_population_count(valid & win)
                    return racc + cnt

                rank = lax.fori_loop(0, nch, body, jnp.zeros((16,), jnp.int32))
                din = jnp.where(rank < K, (c * PP) + g_i * K + rank,
                                jnp.full((16,), trash, jnp.int32))
                plsc.store_scatter(didx_v.at[0], [jnp.full((16,), ll, jnp.int32)],
                                   din, mask=iota == ll)

            pltpu.sync_copy(hbuf.at[pl.ds(l * 16, 16)], out_hbm.at[didx_v.at[0]])


@functools.partial(
    pl.kernel,
    out_type=jax.ShapeDtypeStruct((NC * PP, D), jnp.float32),
    mesh=_mesh,
    scratch_types=[
        pltpu.VMEM((N_NODES,), jnp.float32),
        pltpu.VMEM((N_NODES,), jnp.int32),
        pltpu.VMEM((208,), jnp.int32),
        pltpu.VMEM((NPT, D), jnp.float32),
        pltpu.VMEM((ZB, D), jnp.float32),
        pltpu.VMEM((1, 16), jnp.int32),
        pltpu.SemaphoreType.DMA,
    ],
    compiler_params=_sc_params,
)
def _sc_select(*args):
    _sel_body(*args)


def _head_body(flat_ref, w1_ref, b1_ref, w2_ref, b2_ref, o_ref):
    hid = jnp.maximum(flat_ref[...] @ w1_ref[...] + b1_ref[...][None, :], 0.0)
    o_ref[...] = hid @ w2_ref[...] + b2_ref[...][None, :]


def _head(flat, d1_w, d1_b, d2_w, d2_b):
    return pl.pallas_call(
        _head_body,
        out_shape=jax.ShapeDtypeStruct((flat.shape[0], d2_w.shape[1]), flat.dtype),
    )(flat, d1_w, d1_b, d2_w, d2_b)


def _gcn(h, row, col, norm, W, b):
    m = h @ W
    msg = m[row] * norm[:, None]
    out = jax.ops.segment_sum(msg, col, num_segments=N_NODES)
    return jax.nn.relu(out + b)


def kernel(x, edge_index, edge_weight, node_graph_index, W1, b1, W2, b2, W3, b3,
           conv_w, conv_b, d1_w, d1_b, d2_w, d2_b):
    loop = jnp.arange(N_NODES, dtype=edge_index.dtype)
    row = jnp.concatenate([edge_index[0], loop])
    col = jnp.concatenate([edge_index[1], loop])
    w = jnp.concatenate([edge_weight, jnp.ones((N_NODES,), dtype=edge_weight.dtype)])
    deg = jax.ops.segment_sum(w, row, num_segments=N_NODES)
    dis = deg ** -0.5
    norm = dis[row] * w * dis[col]
    h = _gcn(x, row, col, norm, W1, b1)
    h = _gcn(h, row, col, norm, W2, b2)
    h = _gcn(h, row, col, norm, W3, b3)

    score = h[:, -1]
    starts = jnp.searchsorted(node_graph_index,
                              jnp.arange(201, dtype=jnp.int32)).astype(jnp.int32)
    st_pad = jnp.zeros((208,), jnp.int32).at[:201].set(starts)
    pooledp = _sc_select(score, node_graph_index, st_pad, h)
    pooled = (pooledp[:N_GRAPHS * K] + pooledp[PP:PP + N_GRAPHS * K]
              ).reshape(N_GRAPHS, K, D)

    conv = jax.lax.conv_general_dilated(
        pooled, conv_w, window_strides=(1,), padding='VALID',
        dimension_numbers=('NWC', 'WIO', 'NWC'))
    conv = jax.nn.relu(conv + conv_b)
    flat = conv.reshape(N_GRAPHS, -1)
    return _head(flat, d1_w, d1_b, d2_w, d2_b)
